# baseline (speedup 1.0000x reference)
import jax
import jax.numpy as jnp
from jax import lax
from jax.experimental import pallas as pl
from jax.experimental.pallas import tpu as pltpu
from jax.experimental.pallas import tpu_sc as plsc

B = 16384
D = 64
NC = 2
NS = 16
NW = NC * NS
BPW = B // NW
CHUNK = 128
NCHUNK = BPW // CHUNK
LANES = 16


def _worker_id():
    return lax.axis_index("s") * NC + lax.axis_index("c")


def _partial_body(uidx, pidx, uemb, pemb, ubias, pbias,
                  partials, bsum,
                  idx_u, idx_p, rows_u, rows_p, bu, bp, bs, pacc, sem):
    wid = _worker_id()
    base = wid * BPW
    pltpu.sync_copy(uidx.at[pl.ds(wid * NCHUNK, NCHUNK)], idx_u)
    pltpu.sync_copy(pidx.at[pl.ds(wid * NCHUNK, NCHUNK)], idx_p)
    copies = []
    for j in range(NCHUNK):
        dst = pl.ds(j * CHUNK, CHUNK)
        copies.append(pltpu.async_copy(uemb.at[idx_u.at[j]], rows_u.at[dst], sem))
        copies.append(pltpu.async_copy(pemb.at[idx_p.at[j]], rows_p.at[dst], sem))
        copies.append(pltpu.async_copy(ubias.at[idx_u.at[j]], bu.at[dst], sem))
        copies.append(pltpu.async_copy(pbias.at[idx_p.at[j]], bp.at[dst], sem))
    for c in copies:
        c.wait()

    def dot_body(r, acc):
        s = acc
        for c in range(D // LANES):
            sl = pl.ds(c * LANES, LANES)
            s = s + rows_u[r, sl] * rows_p[r, sl]
        return s

    acc = lax.fori_loop(0, BPW, dot_body, jnp.zeros((LANES,), jnp.float32))
    pacc[...] = acc
    pltpu.sync_copy(pacc, partials.at[wid])

    def bias_body(k, carry):
        sl = pl.ds(k * LANES, LANES)
        bs[sl] = bu[sl] + bp[sl]
        return carry

    lax.fori_loop(0, BPW // LANES, bias_body, 0)
    pltpu.sync_copy(bs, bsum.at[pl.ds(base, BPW)])


def _combine_body(partials, bsum, out, pall, bsv, ob):
    wid = _worker_id()
    base = wid * BPW
    pltpu.sync_copy(partials, pall)
    pltpu.sync_copy(bsum.at[pl.ds(base, BPW)], bsv)

    def sum_body(i, tv):
        return tv + pall[i, :]

    tv = lax.fori_loop(0, NW, sum_body, jnp.zeros((LANES,), jnp.float32))
    total = jnp.sum(tv)

    def sig_body(k, carry):
        sl = pl.ds(k * LANES, LANES)
        x = total + bsv[sl]
        ob[sl] = 1.0 / (1.0 + jnp.exp(-x))
        return carry

    lax.fori_loop(0, BPW // LANES, sig_body, 0)
    pltpu.sync_copy(ob, out.at[pl.ds(base, BPW)])


def kernel(inputs, user_emb, user_bias, place_emb, place_bias):
    u_idx = inputs[:, 0].astype(jnp.int32).reshape(B // CHUNK, CHUNK)
    p_idx = inputs[:, 1].astype(jnp.int32).reshape(B // CHUNK, CHUNK)
    ub = user_bias.reshape(-1)
    pb = place_bias.reshape(-1)

    mesh = plsc.VectorSubcoreMesh(core_axis_name="c", subcore_axis_name="s")
    partial_fn = pl.kernel(
        _partial_body,
        mesh=mesh,
        compiler_params=pltpu.CompilerParams(use_tc_tiling_on_sc=False),
        out_type=(
            jax.ShapeDtypeStruct((NW, LANES), jnp.float32),
            jax.ShapeDtypeStruct((B,), jnp.float32),
        ),
        scratch_types=[
            pltpu.VMEM((NCHUNK, CHUNK), jnp.int32),
            pltpu.VMEM((NCHUNK, CHUNK), jnp.int32),
            pltpu.VMEM((BPW, D), jnp.float32),
            pltpu.VMEM((BPW, D), jnp.float32),
            pltpu.VMEM((BPW,), jnp.float32),
            pltpu.VMEM((BPW,), jnp.float32),
            pltpu.VMEM((BPW,), jnp.float32),
            pltpu.VMEM((LANES,), jnp.float32),
            pltpu.SemaphoreType.DMA,
        ],
    )
    partials, bsum = partial_fn(u_idx, p_idx, user_emb, place_emb, ub, pb)

    combine_fn = pl.kernel(
        _combine_body,
        mesh=plsc.VectorSubcoreMesh(core_axis_name="c", subcore_axis_name="s"),
        compiler_params=pltpu.CompilerParams(needs_layout_passes=False),
        out_type=jax.ShapeDtypeStruct((B,), jnp.float32),
        scratch_types=[
            pltpu.VMEM((NW, LANES), jnp.float32),
            pltpu.VMEM((BPW,), jnp.float32),
            pltpu.VMEM((BPW,), jnp.float32),
        ],
    )
    out = combine_fn(partials, bsum)
    return out.reshape(B, 1)
